# baseline (speedup 1.0000x reference)
import functools

import jax
import jax.numpy as jnp
from jax import lax
from jax.experimental import pallas as pl
from jax.experimental.pallas import tpu as pltpu
from jax.experimental.pallas import tpu_sc as plsc

N = 4096
LANES = 16
NUM_CORES = 2
NUM_SUBCORES = 16
NW = NUM_CORES * NUM_SUBCORES
TOTAL = N * N
WORDS_PER_W = TOTAL // NW
CHUNK = 32768
NCHUNK = WORDS_PER_W // CHUNK
SLICES = CHUNK // LANES

RTOL = 1e-5
ATOL = 1e-7

_mesh = plsc.VectorSubcoreMesh(core_axis_name="c", subcore_axis_name="s")


UNROLL = 16


@functools.partial(
    pl.kernel,
    mesh=_mesh,
    out_type=jax.ShapeDtypeStruct((NW, LANES), jnp.int32),
    scratch_types=[
        pltpu.VMEM((CHUNK,), jnp.float32),
        pltpu.VMEM((CHUNK,), jnp.float32),
        pltpu.VMEM((LANES,), jnp.int32),
        pltpu.SemaphoreType.DMA,
        pltpu.SemaphoreType.DMA,
    ],
)
def _roundtrip_check(x_hbm, out_hbm, buf0, buf1, violbuf, sem0, sem1):
    wid = lax.axis_index("s") * NUM_CORES + lax.axis_index("c")
    base = wid * WORDS_PER_W
    bufs = (buf0, buf1)
    sems = (sem0, sem1)

    pend = pltpu.async_copy(x_hbm.at[pl.ds(base, CHUNK)], bufs[0], sems[0])
    ok = jnp.zeros((LANES,), jnp.int32)
    for c in range(NCHUNK):
        pend.wait()
        if c + 1 < NCHUNK:
            pend = pltpu.async_copy(
                x_hbm.at[pl.ds(base + (c + 1) * CHUNK, CHUNK)],
                bufs[(c + 1) % 2], sems[(c + 1) % 2])
        buf = bufs[c % 2]

        @plsc.parallel_loop(0, CHUNK, LANES, unroll=UNROLL, carry=ok)
        def body(off, acc, buf=buf):
            v = buf[pl.ds(off, LANES)]
            return acc + jnp.where(v == v, 0, 1).astype(jnp.int32)

        ok = body

    violbuf[...] = ok
    pltpu.sync_copy(violbuf, out_hbm.at[wid])


def kernel(x):
    counts = _roundtrip_check(x.reshape(-1))
    return (jnp.sum(counts) == 0).reshape(1)


# speedup vs baseline: 2.0822x; 2.0822x over previous
import functools

import jax
import jax.numpy as jnp
from jax import lax
from jax.experimental import pallas as pl
from jax.experimental.pallas import tpu as pltpu
from jax.experimental.pallas import tpu_sc as plsc

N = 4096
LANES = 16
NUM_CORES = 2
NUM_SUBCORES = 16
NW = NUM_CORES * NUM_SUBCORES
TOTAL = N * N
WORDS_PER_W = TOTAL // NW
CHUNK = 32768
NCHUNK = WORDS_PER_W // CHUNK
SLICES = CHUNK // LANES

RTOL = 1e-5
ATOL = 1e-7

_mesh = plsc.VectorSubcoreMesh(core_axis_name="c", subcore_axis_name="s")


UNROLL = 16


@functools.partial(
    pl.kernel,
    mesh=_mesh,
    out_type=jax.ShapeDtypeStruct((NW, LANES), jnp.int32),
    scratch_types=[
        pltpu.VMEM((CHUNK,), jnp.float32),
        pltpu.VMEM((CHUNK,), jnp.float32),
        pltpu.VMEM((LANES,), jnp.int32),
        pltpu.SemaphoreType.DMA,
        pltpu.SemaphoreType.DMA,
    ],
)
def _roundtrip_check(x_hbm, out_hbm, buf0, buf1, violbuf, sem0, sem1):
    wid = lax.axis_index("s") * NUM_CORES + lax.axis_index("c")
    base = wid * WORDS_PER_W
    bufs = (buf0, buf1)
    sems = (sem0, sem1)

    pend = pltpu.async_copy(x_hbm.at[pl.ds(base, CHUNK)], bufs[0], sems[0])
    ok = jnp.zeros((LANES,), jnp.int32)
    for c in range(NCHUNK):
        pend.wait()
        if c + 1 < NCHUNK:
            pend = pltpu.async_copy(
                x_hbm.at[pl.ds(base + (c + 1) * CHUNK, CHUNK)],
                bufs[(c + 1) % 2], sems[(c + 1) % 2])
        buf = bufs[c % 2]

        def body(i, acc, buf=buf):
            off = i * (LANES * UNROLL)
            for u in range(UNROLL):
                v = buf[pl.ds(off + u * LANES, LANES)]
                acc = acc + jnp.where(v == v, 0, 1).astype(jnp.int32)
            return acc

        ok = lax.fori_loop(0, SLICES // UNROLL, body, ok)

    violbuf[...] = ok
    pltpu.sync_copy(violbuf, out_hbm.at[wid])


def kernel(x):
    counts = _roundtrip_check(x.reshape(-1))
    return (jnp.sum(counts) == 0).reshape(1)


# speedup vs baseline: 3.3803x; 1.6234x over previous
import functools

import jax
import jax.numpy as jnp
from jax import lax
from jax.experimental import pallas as pl
from jax.experimental.pallas import tpu as pltpu
from jax.experimental.pallas import tpu_sc as plsc

N = 4096
LANES = 16
NUM_CORES = 2
NUM_SUBCORES = 16
NW = NUM_CORES * NUM_SUBCORES
ROWS_PER_W = N // NW
CH_ROWS = 8
NCHUNK = ROWS_PER_W // CH_ROWS
ROW_SLICES = N // LANES

UNROLL = 16


@functools.partial(
    pl.kernel,
    mesh=plsc.VectorSubcoreMesh(core_axis_name="c", subcore_axis_name="s"),
    out_type=jax.ShapeDtypeStruct((NW, LANES), jnp.int32),
    scratch_types=[
        pltpu.VMEM((CH_ROWS, N), jnp.float32),
        pltpu.VMEM((CH_ROWS, N), jnp.float32),
        pltpu.VMEM((LANES,), jnp.int32),
        pltpu.SemaphoreType.DMA,
        pltpu.SemaphoreType.DMA,
    ],
)
def _roundtrip_check(x_hbm, out_hbm, buf0, buf1, violbuf, sem0, sem1):
    wid = lax.axis_index("s") * NUM_CORES + lax.axis_index("c")
    base = wid * ROWS_PER_W
    bufs = (buf0, buf1)
    sems = (sem0, sem1)

    pend = pltpu.async_copy(x_hbm.at[pl.ds(base, CH_ROWS)], bufs[0], sems[0])
    viol = jnp.zeros((LANES,), jnp.int32)
    for c in range(NCHUNK):
        pend.wait()
        if c + 1 < NCHUNK:
            pend = pltpu.async_copy(
                x_hbm.at[pl.ds(base + (c + 1) * CH_ROWS, CH_ROWS)],
                bufs[(c + 1) % 2], sems[(c + 1) % 2])
        buf = bufs[c % 2]

        for r in range(CH_ROWS):

            def body(i, acc, buf=buf, r=r):
                off = i * (LANES * UNROLL)
                for u in range(UNROLL):
                    v = buf[r, pl.ds(off + u * LANES, LANES)]
                    acc = acc + jnp.where(v == v, 0, 1).astype(jnp.int32)
                return acc

            viol = lax.fori_loop(0, ROW_SLICES // UNROLL, body, viol)

    violbuf[...] = viol
    pltpu.sync_copy(violbuf, out_hbm.at[wid])


def kernel(x):
    counts = _roundtrip_check(x)
    return (jnp.sum(counts) == 0).reshape(1)
